# baseline (speedup 1.0000x reference)
import functools

import jax
import jax.numpy as jnp
from jax import lax
from jax.experimental import pallas as pl
from jax.experimental.pallas import tpu as pltpu
from jax.experimental.pallas import tpu_sc as plsc

D = 128
NC = 2
NS = 16
NW = NC * NS
CH = 64
K = 13


def _sc_gather(xt, weight):
    fields, batch = xt.shape
    nb_per_w = batch // NW
    cpf = nb_per_w // CH
    mesh = plsc.VectorSubcoreMesh(core_axis_name="c", subcore_axis_name="s")

    @functools.partial(
        pl.kernel,
        mesh=mesh,
        out_type=jax.ShapeDtypeStruct((fields, batch, D), jnp.float32),
        scratch_types=[
            pltpu.VMEM((fields, nb_per_w), jnp.int32),
            pltpu.VMEM((K, CH, D), jnp.float32),
            pltpu.SemaphoreType.DMA,
            pltpu.SemaphoreType.DMA,
        ],
    )
    def k(xt_hbm, w_hbm, out_hbm, idx_v, buf_v, gsem, ssem):
        wid = lax.axis_index("s") * NC + lax.axis_index("c")
        base = wid * nb_per_w
        pltpu.sync_copy(xt_hbm.at[:, pl.ds(base, nb_per_w)], idx_v)

        def start_gather(j, cc, b):
            return pltpu.async_copy(
                w_hbm.at[idx_v.at[j, pl.ds(cc * CH, CH)]], buf_v.at[b], gsem
            )

        def start_scatter(j, cc, b):
            return pltpu.async_copy(
                buf_v.at[b], out_hbm.at[j, pl.ds(base + cc * CH, CH)], ssem
            )

        def wait_scatter(j, cc, b):
            pltpu.make_async_copy(
                buf_v.at[b], out_hbm.at[j, pl.ds(base + cc * CH, CH)], ssem
            ).wait()

        nch = fields * cpf

        gathers = [start_gather(b // cpf, b % cpf, b) for b in range(K)]
        for b in range(K):
            gathers[b].wait()
            start_scatter(b // cpf, b % cpf, b)

        def group(g, carry):
            c0 = g * K
            gathers = []
            for b in range(K):
                cp = c0 - K + b
                wait_scatter(cp // cpf, cp % cpf, b)
                c = c0 + b
                gathers.append(start_gather(c // cpf, c % cpf, b))
            for b in range(K):
                gathers[b].wait()
                c = c0 + b
                start_scatter(c // cpf, c % cpf, b)
            return carry

        lax.fori_loop(1, nch // K, group, 0, unroll=False)

        for b in range(K):
            c = nch - K + b
            wait_scatter(c // cpf, c % cpf, b)

    return k(xt, weight)


def kernel(x, weight):
    batch, fields = x.shape
    assert batch % (NW * CH) == 0 and (fields * batch // NW // CH) % K == 0
    xt = jnp.transpose(x.astype(jnp.int32))
    out_t = _sc_gather(xt, weight)
    return jnp.transpose(out_t, (1, 0, 2))


# speedup vs baseline: 1.0134x; 1.0134x over previous
import functools

import jax
import jax.numpy as jnp
from jax import lax
from jax.experimental import pallas as pl
from jax.experimental.pallas import tpu as pltpu
from jax.experimental.pallas import tpu_sc as plsc

D = 128
NC = 2
NS = 16
NW = NC * NS
CH = 128
K = 4


def _sc_gather(xt, weight):
    fields, batch = xt.shape
    nb_per_w = batch // NW
    cpf = nb_per_w // CH
    mesh = plsc.VectorSubcoreMesh(core_axis_name="c", subcore_axis_name="s")

    @functools.partial(
        pl.kernel,
        mesh=mesh,
        out_type=jax.ShapeDtypeStruct((fields, batch, D), jnp.float32),
        scratch_types=[
            pltpu.VMEM((fields, nb_per_w), jnp.int32),
            pltpu.VMEM((K, CH, D), jnp.float32),
            pltpu.SemaphoreType.DMA,
            pltpu.SemaphoreType.DMA,
        ],
    )
    def k(xt_hbm, w_hbm, out_hbm, idx_v, buf_v, gsem, ssem):
        wid = lax.axis_index("s") * NC + lax.axis_index("c")
        base = wid * nb_per_w
        pltpu.sync_copy(xt_hbm.at[:, pl.ds(base, nb_per_w)], idx_v)

        def start_gather(j, cc, b):
            return pltpu.async_copy(
                w_hbm.at[idx_v.at[j, pl.ds(cc * CH, CH)]], buf_v.at[b], gsem
            )

        def start_scatter(j, cc, b):
            return pltpu.async_copy(
                buf_v.at[b], out_hbm.at[j, pl.ds(base + cc * CH, CH)], ssem
            )

        def wait_scatter(j, cc, b):
            pltpu.make_async_copy(
                buf_v.at[b], out_hbm.at[j, pl.ds(base + cc * CH, CH)], ssem
            ).wait()

        gathers = [start_gather(0, b, b) for b in range(K)]
        for b in range(K):
            gathers[b].wait()
            start_scatter(0, b, b)

        def field(j, carry):
            gathers = []
            for b in range(K):
                wait_scatter(j - 1, b, b)
                gathers.append(start_gather(j, b, b))
            for b in range(K):
                gathers[b].wait()
                start_scatter(j, b, b)
            return carry

        lax.fori_loop(1, fields, field, 0, unroll=False)

        for b in range(K):
            wait_scatter(fields - 1, b, b)

    return k(xt, weight)


def kernel(x, weight):
    batch, fields = x.shape
    assert batch % (NW * CH * K) == 0 and batch // NW // CH == K
    xt = jnp.transpose(x.astype(jnp.int32))
    out_t = _sc_gather(xt, weight)
    return jnp.transpose(out_t, (1, 0, 2))
